# baseline (speedup 1.0000x reference)
import functools

import jax
import jax.numpy as jnp
from jax import lax
from jax.experimental import pallas as pl
from jax.experimental.pallas import tpu as pltpu
from jax.experimental.pallas import tpu_sc as plsc

_B, _S, _NX, _NE, _K = 4, 2048, 4096, 64, 8
_S_BLK = 512
_N_SBLK = _S // _S_BLK
_L = 16
_NCH = _NE // _L


def _router_scores_kernel(x_ref, w_ref, b_ref, scores_ref, acc_ref):
    bi = pl.program_id(0)
    j = pl.program_id(1)

    @pl.when(jnp.logical_and(bi == 0, j == 0))
    def _init():
        acc_ref[...] = jnp.zeros_like(acc_ref)

    ones = jnp.ones((1, _S_BLK), jnp.float32)
    partial = jnp.dot(ones, x_ref[0], precision=lax.Precision.DEFAULT,
                      preferred_element_type=jnp.float32)
    acc_ref[pl.ds(bi, 1), :] += partial

    @pl.when(jnp.logical_and(bi == _B - 1, j == _N_SBLK - 1))
    def _finalize():
        xs = acc_ref[...]
        xh = xs.astype(jnp.bfloat16)
        xl = (xs - xh.astype(jnp.float32)).astype(jnp.bfloat16)
        stacked = jnp.concatenate([xh, xl], axis=0)
        r = jnp.dot(stacked, w_ref[...],
                    preferred_element_type=jnp.float32)
        scores_ref[...] = (r[:_B] + r[_B:]) * (1.0 / _S) + b_ref[...]


def _tc_scores(x, W, b):
    b2 = b.reshape(1, _NE)
    wb16 = W.astype(jnp.bfloat16)
    return pl.pallas_call(
        _router_scores_kernel,
        grid=(_B, _N_SBLK),
        in_specs=[
            pl.BlockSpec((1, _S_BLK, _NX), lambda bi, j: (bi, j, 0)),
            pl.BlockSpec((_NX, _NE), lambda bi, j: (0, 0)),
            pl.BlockSpec((1, _NE), lambda bi, j: (0, 0)),
        ],
        out_specs=pl.BlockSpec((_B, _NE), lambda bi, j: (0, 0)),
        out_shape=jax.ShapeDtypeStruct((_B, _NE), jnp.float32),
        scratch_shapes=[pltpu.VMEM((_B, _NX), jnp.float32)],
        compiler_params=pltpu.CompilerParams(
            dimension_semantics=("arbitrary", "arbitrary"),
        ),
    )(x, wb16, b2)


def _sc_topk_body(scores_hbm, vals_hbm, idx_hbm, row_v, vals_v, idx_v):
    wid = lax.axis_index("s") * 2 + lax.axis_index("c")

    def _bmax(v):
        return plsc.cummax(lax.rev(plsc.cummax(v), (0,)))

    @pl.when(wid < _B)
    def _work():
        pltpu.sync_copy(scores_hbm.at[wid], row_v)
        iota = jnp.arange(_L, dtype=jnp.int32)
        ch = [row_v[pl.ds(c * _L, _L)] for c in range(_NCH)]
        q = ch[0]
        for c in range(1, _NCH):
            q = jnp.maximum(q, ch[c])
        m = _bmax(q)
        es = [jnp.exp(x - m) for x in ch]
        t = es[0]
        for c in range(1, _NCH):
            t = t + es[c]
        tot = plsc.cummax(lax.rev(plsc.cumsum(t), (0,)))
        ps = [e / tot for e in es]
        vals_acc = jnp.zeros((_L,), jnp.float32)
        idx_acc = jnp.zeros((_L,), jnp.int32)
        for k in range(_K):
            q = ps[0]
            for c in range(1, _NCH):
                q = jnp.maximum(q, ps[c])
            mk = _bmax(q)
            cand = jnp.where(ps[0] == mk, iota, _NE)
            for c in range(1, _NCH):
                cand = jnp.minimum(
                    cand, jnp.where(ps[c] == mk, iota + _L * c, _NE))
            ik = -_bmax(-cand)
            vals_acc = jnp.where(iota == k, mk, vals_acc)
            idx_acc = jnp.where(iota == k, ik, idx_acc)
            ps = [jnp.where(iota + _L * c == ik, -jnp.inf, ps[c])
                  for c in range(_NCH)]
        vals_v[...] = vals_acc
        idx_v[...] = idx_acc
        pltpu.sync_copy(vals_v, vals_hbm.at[wid])
        pltpu.sync_copy(idx_v, idx_hbm.at[wid])


def _sc_topk(scores):
    mesh = plsc.VectorSubcoreMesh(core_axis_name="c", subcore_axis_name="s")
    fn = functools.partial(
        pl.kernel, mesh=mesh,
        out_type=[jax.ShapeDtypeStruct((_B, _L), jnp.float32),
                  jax.ShapeDtypeStruct((_B, _L), jnp.int32)],
        scratch_types=[pltpu.VMEM((_NE,), jnp.float32),
                       pltpu.VMEM((_L,), jnp.float32),
                       pltpu.VMEM((_L,), jnp.int32)],
        compiler_params=pltpu.CompilerParams(needs_layout_passes=False),
    )(_sc_topk_body)
    return fn(scores)


@jax.jit
def kernel(x, W, b):
    scores = _tc_scores(x, W, b)
    vals16, idx16 = _sc_topk(scores)
    return vals16[:, :_K], idx16[:, :_K]


# speedup vs baseline: 1.3374x; 1.3374x over previous
import jax
import jax.numpy as jnp
from jax import lax
from jax.experimental import pallas as pl
from jax.experimental.pallas import tpu as pltpu

_B, _S, _NX, _NE, _K = 4, 2048, 4096, 64, 8
_S_BLK = 512
_N_SBLK = _S // _S_BLK


def _router_kernel(x_ref, w_ref, b_ref, vals_ref, idx_ref, acc_ref):
    bi = pl.program_id(0)
    j = pl.program_id(1)

    @pl.when(jnp.logical_and(bi == 0, j == 0))
    def _init():
        acc_ref[...] = jnp.zeros_like(acc_ref)

    ones = jnp.ones((1, _S_BLK), jnp.float32)
    partial = jnp.dot(ones, x_ref[0], precision=lax.Precision.DEFAULT,
                      preferred_element_type=jnp.float32)
    acc_ref[pl.ds(bi, 1), :] += partial

    @pl.when(jnp.logical_and(bi == _B - 1, j == _N_SBLK - 1))
    def _finalize():
        xs = acc_ref[...]
        xh = xs.astype(jnp.bfloat16)
        xl = (xs - xh.astype(jnp.float32)).astype(jnp.bfloat16)
        stacked = jnp.concatenate([xh, xl], axis=0)
        r = jnp.dot(stacked, w_ref[...],
                    preferred_element_type=jnp.float32)
        scores = (r[:_B] + r[_B:]) * (1.0 / _S) + b_ref[...]
        m = jnp.max(scores, axis=1, keepdims=True)
        e = jnp.exp(scores - m)
        p = e / jnp.sum(e, axis=1, keepdims=True)

        iota = lax.broadcasted_iota(jnp.int32, (_B, _NE), 1)
        s = p
        for k in range(_K):
            mk = jnp.max(s, axis=1, keepdims=True)
            ik = jnp.min(jnp.where(s == mk, iota, _NE),
                         axis=1, keepdims=True)
            vals_ref[:, k:k + 1] = mk
            idx_ref[:, k:k + 1] = ik
            s = jnp.where(iota == ik, -jnp.inf, s)


@jax.jit
def kernel(x, W, b):
    b2 = b.reshape(1, _NE)
    wb16 = W.astype(jnp.bfloat16)
    vals, idx = pl.pallas_call(
        _router_kernel,
        grid=(_B, _N_SBLK),
        in_specs=[
            pl.BlockSpec((1, _S_BLK, _NX), lambda bi, j: (bi, j, 0)),
            pl.BlockSpec((_NX, _NE), lambda bi, j: (0, 0)),
            pl.BlockSpec((1, _NE), lambda bi, j: (0, 0)),
        ],
        out_specs=[
            pl.BlockSpec((_B, _K), lambda bi, j: (0, 0)),
            pl.BlockSpec((_B, _K), lambda bi, j: (0, 0)),
        ],
        out_shape=[
            jax.ShapeDtypeStruct((_B, _K), jnp.float32),
            jax.ShapeDtypeStruct((_B, _K), jnp.int32),
        ],
        scratch_shapes=[pltpu.VMEM((_B, _NX), jnp.float32)],
        compiler_params=pltpu.CompilerParams(
            dimension_semantics=("arbitrary", "arbitrary"),
        ),
    )(x, wb16, b2)
    return vals, idx
